# baseline (speedup 1.0000x reference)
import functools

import jax
import jax.numpy as jnp
from jax import lax
from jax.experimental import pallas as pl
from jax.experimental.pallas import tpu as pltpu
from jax.experimental.pallas import tpu_sc as plsc

N = 10000
E = 160000
D = 256
H = 128
G = 128
NS = 16
NC = 2

CHUNK = 128
NBUF = 3
IDEPTH = 4
GROUP = 12
CH = 84
EDGES_PER_TILE = CH * CHUNK
E_PAD = NS * EDGES_PER_TILE
NGROUPS = CH // GROUP
NROWS = N + 8

RPT = 624
RPT_LAST = 640
ROW0_LAST = 15 * RPT

PCH = 5
FULL_CHUNKS = N // CHUNK
TAIL = N - FULL_CHUNKS * CHUNK
GROWS = G + 8
GPT = G // NS

_mesh = plsc.VectorSubcoreMesh(core_axis_name="c", subcore_axis_name="s")


@functools.partial(
    pl.kernel,
    out_type=jax.ShapeDtypeStruct((NC, N, H), jnp.float32),
    mesh=_mesh,
    scratch_types=[
        pltpu.VMEM((IDEPTH, CHUNK), jnp.int32),
        pltpu.VMEM((IDEPTH, CHUNK), jnp.int32),
        pltpu.VMEM((NBUF, CHUNK, H), jnp.float32),
        pltpu.VMEM_SHARED((NROWS, H), jnp.float32),
        pltpu.SemaphoreType.DMA((NBUF,)),
        pltpu.SemaphoreType.DMA((NBUF,)),
        pltpu.SemaphoreType.DMA((IDEPTH,)),
    ],
)
def _sc_message(h2, src_f, dst_f, pre2, srcv, dstv, rows, acc,
                gsem, ssem, isem):
    c = lax.axis_index("c")
    s = lax.axis_index("s")
    table = h2.at[c]
    tile_base = s * EDGES_PER_TILE

    @pl.when(s < 15)
    def _():
        row0 = pl.multiple_of(s * RPT, 8)
        pltpu.sync_copy(table.at[pl.ds(row0, RPT)],
                        acc.at[pl.ds(row0, RPT)])

    @pl.when(s == 15)
    def _():
        pltpu.sync_copy(table.at[pl.ds(ROW0_LAST, RPT_LAST)],
                        acc.at[pl.ds(ROW0_LAST, RPT_LAST)])

    def fire_idx(i, b):
        off = pl.multiple_of(tile_base + i * CHUNK, 8)
        pltpu.async_copy(src_f.at[pl.ds(off, CHUNK)], srcv.at[b], isem.at[b])
        pltpu.async_copy(dst_f.at[pl.ds(off, CHUNK)], dstv.at[b], isem.at[b])

    def wait_idx(b):
        pltpu.make_async_copy(src_f.at[pl.ds(0, CHUNK)], srcv.at[b],
                              isem.at[b]).wait()
        pltpu.make_async_copy(dst_f.at[pl.ds(0, CHUNK)], dstv.at[b],
                              isem.at[b]).wait()

    def fire_gather(bi, br):
        pltpu.async_copy(table.at[srcv.at[bi]], rows.at[br], gsem.at[br])

    def wait_gather(bi, br):
        pltpu.make_async_copy(table.at[srcv.at[bi]], rows.at[br],
                              gsem.at[br]).wait()

    def fire_scatter(bi, br):
        pltpu.async_copy(rows.at[br], acc.at[dstv.at[bi]], ssem.at[br],
                         add=True)

    def wait_scatter(bi, br):
        pltpu.make_async_copy(rows.at[br], acc.at[dstv.at[bi]],
                              ssem.at[br]).wait()

    for j in range(IDEPTH - 1):
        fire_idx(j, j)
    plsc.subcore_barrier()
    for j in range(2):
        wait_idx(j)
        fire_gather(j, j)

    @pl.loop(0, NGROUPS)
    def _(g):
        for b in range(GROUP):
            i = g * GROUP + b

            @pl.when(i >= 1)
            def _():
                wait_scatter((b - 1) % IDEPTH, (b - 1) % NBUF)

            @pl.when(i + IDEPTH - 1 < CH)
            def _():
                fire_idx(i + IDEPTH - 1, (b - 1) % IDEPTH)

            wait_gather(b % IDEPTH, b % NBUF)
            fire_scatter(b % IDEPTH, b % NBUF)

            @pl.when(i + 2 < CH)
            def _():
                wait_idx((b + 2) % IDEPTH)
                fire_gather((b + 2) % IDEPTH, (b + 2) % NBUF)

    wait_scatter((CH - 1) % IDEPTH, (CH - 1) % NBUF)
    plsc.subcore_barrier()

    @pl.when(s < 15)
    def _():
        row0 = pl.multiple_of(s * RPT, 8)
        pltpu.sync_copy(acc.at[pl.ds(row0, RPT)],
                        pre2.at[c, pl.ds(row0, RPT)])

    @pl.when(s == 15)
    def _():
        pltpu.sync_copy(acc.at[pl.ds(ROW0_LAST, RPT_LAST)],
                        pre2.at[c, pl.ds(ROW0_LAST, RPT_LAST)])


@functools.partial(
    pl.kernel,
    out_type=jax.ShapeDtypeStruct((NC, G, H), jnp.float32),
    mesh=_mesh,
    scratch_types=[
        pltpu.VMEM((PCH, CHUNK), jnp.int32),
        pltpu.VMEM((CHUNK, H), jnp.float32),
        pltpu.VMEM((GPT, H), jnp.float32),
        pltpu.VMEM_SHARED((GROWS, H), jnp.float32),
    ],
)
def _sc_pool(h2, batch_r, pooled, bidx, pbuf, zbuf, acc):
    c = lax.axis_index("c")
    s = lax.axis_index("s")

    @pl.loop(0, GPT)
    def _(r):
        for j in range(H // 16):
            zbuf[r, pl.ds(j * 16, 16)] = jnp.zeros((16,), jnp.float32)
    pltpu.sync_copy(zbuf, acc.at[pl.ds(s * GPT, GPT)])
    @pl.when(s == 0)
    def _():
        pltpu.sync_copy(zbuf.at[pl.ds(0, GROWS - G)],
                        acc.at[pl.ds(G, GROWS - G)])
    pltpu.sync_copy(batch_r.at[s], bidx)
    plsc.subcore_barrier()

    for k in range(PCH):
        ci = s * PCH + k

        @pl.when(ci < FULL_CHUNKS)
        def _():
            off = pl.multiple_of(ci * CHUNK, 8)
            pltpu.sync_copy(h2.at[c, pl.ds(off, CHUNK)], pbuf)

        @pl.when(ci == FULL_CHUNKS)
        def _():
            pltpu.sync_copy(h2.at[c, pl.ds(FULL_CHUNKS * CHUNK, TAIL)],
                            pbuf.at[pl.ds(0, TAIL)])

        pltpu.sync_copy(pbuf, acc.at[bidx.at[k]], add=True)

    plsc.subcore_barrier()
    pltpu.sync_copy(acc.at[pl.ds(s * GPT, GPT)],
                    pooled.at[c, pl.ds(s * GPT, GPT)])


BN = 1000
NB = N // BN


def _tc_mlp_body(pre_ref, w1_ref, b1_ref, w2_ref, b2_ref, out_ref):
    x = jnp.concatenate([pre_ref[0], pre_ref[1]], axis=1)
    t = jnp.maximum(
        jnp.dot(x, w1_ref[...], preferred_element_type=jnp.float32)
        + b1_ref[...], 0.0)
    y = jnp.maximum(
        jnp.dot(t, w2_ref[...], preferred_element_type=jnp.float32)
        + b2_ref[...], 0.0)
    out_ref[0] = y[:, :H]
    out_ref[1] = y[:, H:]


def _tc_mlp(pre2, w1, b1, w2, b2):
    return pl.pallas_call(
        _tc_mlp_body,
        grid=(NB,),
        in_specs=[
            pl.BlockSpec((NC, BN, H), lambda i: (0, i, 0)),
            pl.BlockSpec((D, D), lambda i: (0, 0)),
            pl.BlockSpec((1, D), lambda i: (0, 0)),
            pl.BlockSpec((D, D), lambda i: (0, 0)),
            pl.BlockSpec((1, D), lambda i: (0, 0)),
        ],
        out_specs=pl.BlockSpec((NC, BN, H), lambda i: (0, i, 0)),
        out_shape=jax.ShapeDtypeStruct((NC, N, H), jnp.float32),
    )(pre2, w1, b1, w2, b2)


def _tc_finish_body(pooled_ref, batch_ref, out_ref):
    b = batch_ref[...]
    gi = lax.broadcasted_iota(jnp.int32, (1, G), 1)
    cnt = jnp.sum((b == gi).astype(jnp.float32), axis=0)
    denom = jnp.maximum(cnt, 1.0)[:, None]
    hg = jnp.concatenate([pooled_ref[0], pooled_ref[1]], axis=1)
    out_ref[...] = hg / denom


def _tc_finish(pooled, batch_2d):
    return pl.pallas_call(
        _tc_finish_body,
        out_shape=jax.ShapeDtypeStruct((G, D), jnp.float32),
    )(pooled, batch_2d)


def kernel(x, edge_index, batch, W1_0, b1_0, W2_0, b2_0, W1_1, b1_1, W2_1,
           b2_1, W1_2, b1_2, W2_2, b2_2):
    src = edge_index[0]
    dst = edge_index[1]
    pad = E_PAD - E
    src_f = jnp.concatenate([src, jnp.zeros((pad,), jnp.int32)])
    dst_f = jnp.concatenate([dst, jnp.full((pad,), N, jnp.int32)])

    batch_r = jnp.concatenate(
        [batch, jnp.full((NS * PCH * CHUNK - N,), G, jnp.int32)]
    ).reshape(NS, PCH, CHUNK)
    batch_2d = jnp.concatenate(
        [batch, jnp.full((80 * 128 - N,), -1, jnp.int32)]).reshape(80 * 128, 1)

    h2 = jnp.stack([x[:, :H], x[:, H:]])
    weights = [(W1_0, b1_0, W2_0, b2_0), (W1_1, b1_1, W2_1, b2_1),
               (W1_2, b1_2, W2_2, b2_2)]

    for (w1, b1, w2, b2) in weights:
        pre2 = _sc_message(h2, src_f, dst_f)
        h2 = _tc_mlp(pre2, w1, b1.reshape(1, D), w2, b2.reshape(1, D))

    pooled = _sc_pool(h2, batch_r)
    return _tc_finish(pooled, batch_2d)


# speedup vs baseline: 4.1394x; 4.1394x over previous
import functools

import jax
import jax.numpy as jnp
from jax import lax
from jax.experimental import pallas as pl
from jax.experimental.pallas import tpu as pltpu
from jax.experimental.pallas import tpu_sc as plsc

N = 10000
E = 160000
D = 256
H = 128
G = 128
NS = 16
NC = 2

CHUNK = 112
NBUF = 3
IDEPTH = 9
GROUP = 9
CH = 90
EDGES_PER_TILE = CH * CHUNK
E_PAD = NS * EDGES_PER_TILE
NGROUPS = CH // GROUP
NROWS = N + 8

RPT = 624
RPT_LAST = 640
ROW0_LAST = 15 * RPT

PCH = 5
PCHUNK = 128
FULL_CHUNKS = N // PCHUNK
TAIL = N - FULL_CHUNKS * PCHUNK
GROWS = G + 8
GPT = G // NS

_mesh = plsc.VectorSubcoreMesh(core_axis_name="c", subcore_axis_name="s")


@functools.partial(
    pl.kernel,
    out_type=jax.ShapeDtypeStruct((NC, N, H), jnp.float32),
    mesh=_mesh,
    scratch_types=[
        pltpu.VMEM((IDEPTH, CHUNK), jnp.int32),
        pltpu.VMEM((IDEPTH, CHUNK), jnp.int32),
        pltpu.VMEM((NBUF, CHUNK, H), jnp.float32),
        pltpu.VMEM_SHARED((NROWS, H), jnp.float32),
        pltpu.SemaphoreType.DMA((NBUF,)),
        pltpu.SemaphoreType.DMA((NBUF,)),
        pltpu.SemaphoreType.DMA((IDEPTH,)),
    ],
)
def _sc_message(h2, src_f, dst_f, pre2, srcv, dstv, rows, acc,
                gsem, ssem, isem):
    c = lax.axis_index("c")
    s = lax.axis_index("s")
    table = h2.at[c]
    tile_base = s * EDGES_PER_TILE

    @pl.when(s < 15)
    def _():
        row0 = pl.multiple_of(s * RPT, 8)
        pltpu.sync_copy(table.at[pl.ds(row0, RPT)],
                        acc.at[pl.ds(row0, RPT)])

    @pl.when(s == 15)
    def _():
        pltpu.sync_copy(table.at[pl.ds(ROW0_LAST, RPT_LAST)],
                        acc.at[pl.ds(ROW0_LAST, RPT_LAST)])

    def fire_idx(i, b):
        off = pl.multiple_of(tile_base + i * CHUNK, 8)
        pltpu.async_copy(src_f.at[pl.ds(off, CHUNK)], srcv.at[b], isem.at[b])
        pltpu.async_copy(dst_f.at[pl.ds(off, CHUNK)], dstv.at[b], isem.at[b])

    def wait_idx(b):
        pltpu.make_async_copy(src_f.at[pl.ds(0, CHUNK)], srcv.at[b],
                              isem.at[b]).wait()
        pltpu.make_async_copy(dst_f.at[pl.ds(0, CHUNK)], dstv.at[b],
                              isem.at[b]).wait()

    def fire_gather(bi, br):
        pltpu.async_copy(table.at[srcv.at[bi]], rows.at[br], gsem.at[br])

    def wait_gather(bi, br):
        pltpu.make_async_copy(table.at[srcv.at[bi]], rows.at[br],
                              gsem.at[br]).wait()

    def fire_scatter(bi, br):
        pltpu.async_copy(rows.at[br], acc.at[dstv.at[bi]], ssem.at[br],
                         add=True)

    def wait_scatter(bi, br):
        pltpu.make_async_copy(rows.at[br], acc.at[dstv.at[bi]],
                              ssem.at[br]).wait()

    for j in range(IDEPTH - 1):
        fire_idx(j, j)
    plsc.subcore_barrier()
    for j in range(2):
        wait_idx(j)
        fire_gather(j, j)

    @pl.loop(0, NGROUPS)
    def _(g):
        for b in range(GROUP):
            i = g * GROUP + b

            @pl.when(i >= 1)
            def _():
                wait_scatter((b - 1) % IDEPTH, (b - 1) % NBUF)

            @pl.when(i + IDEPTH - 1 < CH)
            def _():
                fire_idx(i + IDEPTH - 1, (b - 1) % IDEPTH)

            wait_gather(b % IDEPTH, b % NBUF)
            fire_scatter(b % IDEPTH, b % NBUF)

            @pl.when(i + 2 < CH)
            def _():
                wait_idx((b + 2) % IDEPTH)
                fire_gather((b + 2) % IDEPTH, (b + 2) % NBUF)

    wait_scatter((CH - 1) % IDEPTH, (CH - 1) % NBUF)
    plsc.subcore_barrier()

    @pl.when(s < 15)
    def _():
        row0 = pl.multiple_of(s * RPT, 8)
        pltpu.sync_copy(acc.at[pl.ds(row0, RPT)],
                        pre2.at[c, pl.ds(row0, RPT)])

    @pl.when(s == 15)
    def _():
        pltpu.sync_copy(acc.at[pl.ds(ROW0_LAST, RPT_LAST)],
                        pre2.at[c, pl.ds(ROW0_LAST, RPT_LAST)])


@functools.partial(
    pl.kernel,
    out_type=jax.ShapeDtypeStruct((NC, G, H), jnp.float32),
    mesh=_mesh,
    scratch_types=[
        pltpu.VMEM((PCH, PCHUNK), jnp.int32),
        pltpu.VMEM((PCHUNK, H), jnp.float32),
        pltpu.VMEM((GPT, H), jnp.float32),
        pltpu.VMEM_SHARED((GROWS, H), jnp.float32),
    ],
)
def _sc_pool(h2, batch_r, pooled, bidx, pbuf, zbuf, acc):
    c = lax.axis_index("c")
    s = lax.axis_index("s")

    @pl.loop(0, GPT)
    def _(r):
        for j in range(H // 16):
            zbuf[r, pl.ds(j * 16, 16)] = jnp.zeros((16,), jnp.float32)
    pltpu.sync_copy(zbuf, acc.at[pl.ds(s * GPT, GPT)])
    @pl.when(s == 0)
    def _():
        pltpu.sync_copy(zbuf.at[pl.ds(0, GROWS - G)],
                        acc.at[pl.ds(G, GROWS - G)])
    pltpu.sync_copy(batch_r.at[s], bidx)
    plsc.subcore_barrier()

    for k in range(PCH):
        ci = s * PCH + k

        @pl.when(ci < FULL_CHUNKS)
        def _():
            off = pl.multiple_of(ci * PCHUNK, 8)
            pltpu.sync_copy(h2.at[c, pl.ds(off, PCHUNK)], pbuf)

        @pl.when(ci == FULL_CHUNKS)
        def _():
            pltpu.sync_copy(h2.at[c, pl.ds(FULL_CHUNKS * PCHUNK, TAIL)],
                            pbuf.at[pl.ds(0, TAIL)])

        pltpu.sync_copy(pbuf, acc.at[bidx.at[k]], add=True)

    plsc.subcore_barrier()
    pltpu.sync_copy(acc.at[pl.ds(s * GPT, GPT)],
                    pooled.at[c, pl.ds(s * GPT, GPT)])


BN = 1000
NB = N // BN


def _tc_mlp_body(pre_ref, w1_ref, b1_ref, w2_ref, b2_ref, out_ref):
    x = jnp.concatenate([pre_ref[0], pre_ref[1]], axis=1)
    t = jnp.maximum(
        jnp.dot(x, w1_ref[...], preferred_element_type=jnp.float32)
        + b1_ref[...], 0.0)
    y = jnp.maximum(
        jnp.dot(t, w2_ref[...], preferred_element_type=jnp.float32)
        + b2_ref[...], 0.0)
    out_ref[0] = y[:, :H]
    out_ref[1] = y[:, H:]


def _tc_mlp(pre2, w1, b1, w2, b2):
    return pl.pallas_call(
        _tc_mlp_body,
        grid=(NB,),
        in_specs=[
            pl.BlockSpec((NC, BN, H), lambda i: (0, i, 0)),
            pl.BlockSpec((D, D), lambda i: (0, 0)),
            pl.BlockSpec((1, D), lambda i: (0, 0)),
            pl.BlockSpec((D, D), lambda i: (0, 0)),
            pl.BlockSpec((1, D), lambda i: (0, 0)),
        ],
        out_specs=pl.BlockSpec((NC, BN, H), lambda i: (0, i, 0)),
        out_shape=jax.ShapeDtypeStruct((NC, N, H), jnp.float32),
    )(pre2, w1, b1, w2, b2)


def _tc_finish_body(pooled_ref, batch_ref, out_ref):
    b = batch_ref[...]
    gi = lax.broadcasted_iota(jnp.int32, (1, G), 1)
    cnt = jnp.sum((b == gi).astype(jnp.float32), axis=0)
    denom = jnp.maximum(cnt, 1.0)[:, None]
    hg = jnp.concatenate([pooled_ref[0], pooled_ref[1]], axis=1)
    out_ref[...] = hg / denom


def _tc_finish(pooled, batch_2d):
    return pl.pallas_call(
        _tc_finish_body,
        out_shape=jax.ShapeDtypeStruct((G, D), jnp.float32),
    )(pooled, batch_2d)


def kernel(x, edge_index, batch, W1_0, b1_0, W2_0, b2_0, W1_1, b1_1, W2_1,
           b2_1, W1_2, b1_2, W2_2, b2_2):
    src = edge_index[0]
    dst = edge_index[1]
    pad = E_PAD - E
    src_f = jnp.concatenate([src, jnp.zeros((pad,), jnp.int32)])
    dst_f = jnp.concatenate([dst, jnp.full((pad,), N, jnp.int32)])

    batch_r = jnp.concatenate(
        [batch, jnp.full((NS * PCH * PCHUNK - N,), G, jnp.int32)]
    ).reshape(NS, PCH, PCHUNK)
    batch_2d = jnp.concatenate(
        [batch, jnp.full((80 * 128 - N,), -1, jnp.int32)]).reshape(80 * 128, 1)

    h2 = jnp.stack([x[:, :H], x[:, H:]])
    weights = [(W1_0, b1_0, W2_0, b2_0), (W1_1, b1_1, W2_1, b2_1),
               (W1_2, b1_2, W2_2, b2_2)]

    for (w1, b1, w2, b2) in weights:
        pre2 = _sc_message(h2, src_f, dst_f)
        h2 = _tc_mlp(pre2, w1, b1.reshape(1, D), w2, b2.reshape(1, D))

    pooled = _sc_pool(h2, batch_r)
    return _tc_finish(pooled, batch_2d)


# speedup vs baseline: 4.5376x; 1.0962x over previous
import functools

import jax
import jax.numpy as jnp
from jax import lax
from jax.experimental import pallas as pl
from jax.experimental.pallas import tpu as pltpu
from jax.experimental.pallas import tpu_sc as plsc

N = 10000
E = 160000
D = 256
H = 128
G = 128
NS = 16
NC = 2

CHUNK = 120
NBUF = 3
IDEPTH = 12
GROUP = 12
CH = 84
EDGES_PER_TILE = CH * CHUNK
E_PAD = NS * EDGES_PER_TILE
NGROUPS = CH // GROUP
NROWS = N + 8

RPT = 624
RPT_LAST = 640
ROW0_LAST = 15 * RPT

PCH = 5
PCHUNK = 128
FULL_CHUNKS = N // PCHUNK
TAIL = N - FULL_CHUNKS * PCHUNK
GROWS = G + 8
GPT = G // NS

_mesh = plsc.VectorSubcoreMesh(core_axis_name="c", subcore_axis_name="s")


@functools.partial(
    pl.kernel,
    out_type=jax.ShapeDtypeStruct((NC, N, H), jnp.float32),
    mesh=_mesh,
    scratch_types=[
        pltpu.VMEM((IDEPTH, 2, CHUNK), jnp.int32),
        pltpu.VMEM((NBUF, CHUNK, H), jnp.float32),
        pltpu.VMEM_SHARED((NROWS, H), jnp.float32),
        pltpu.SemaphoreType.DMA((NBUF,)),
        pltpu.SemaphoreType.DMA((NBUF,)),
        pltpu.SemaphoreType.DMA((IDEPTH,)),
    ],
)
def _sc_message(h2, edge_p, pre2, idxv, rows, acc, gsem, ssem, isem):
    c = lax.axis_index("c")
    s = lax.axis_index("s")
    table = h2.at[c]

    @pl.when(s < 15)
    def _():
        row0 = pl.multiple_of(s * RPT, 8)
        pltpu.sync_copy(table.at[pl.ds(row0, RPT)],
                        acc.at[pl.ds(row0, RPT)])

    @pl.when(s == 15)
    def _():
        pltpu.sync_copy(table.at[pl.ds(ROW0_LAST, RPT_LAST)],
                        acc.at[pl.ds(ROW0_LAST, RPT_LAST)])

    def fire_idx(i, b):
        pltpu.async_copy(edge_p.at[s, i], idxv.at[b], isem.at[b])

    def wait_idx(b):
        pltpu.make_async_copy(edge_p.at[0, 0], idxv.at[b],
                              isem.at[b]).wait()

    def fire_gather(bi, br):
        pltpu.async_copy(table.at[idxv.at[bi, 0]], rows.at[br], gsem.at[br])

    def wait_gather(bi, br):
        pltpu.make_async_copy(table.at[idxv.at[bi, 0]], rows.at[br],
                              gsem.at[br]).wait()

    def fire_scatter(bi, br):
        pltpu.async_copy(rows.at[br], acc.at[idxv.at[bi, 1]], ssem.at[br],
                         add=True)

    def wait_scatter(bi, br):
        pltpu.make_async_copy(rows.at[br], acc.at[idxv.at[bi, 1]],
                              ssem.at[br]).wait()

    for j in range(IDEPTH - 1):
        fire_idx(j, j)
    plsc.subcore_barrier()
    for j in range(2):
        wait_idx(j)
        fire_gather(j, j)

    @pl.loop(0, NGROUPS)
    def _(g):
        for b in range(GROUP):
            i = g * GROUP + b

            @pl.when(i >= 1)
            def _():
                wait_scatter((b - 1) % IDEPTH, (b - 1) % NBUF)

            @pl.when(i + IDEPTH - 1 < CH)
            def _():
                fire_idx(i + IDEPTH - 1, (b - 1) % IDEPTH)

            wait_gather(b % IDEPTH, b % NBUF)
            fire_scatter(b % IDEPTH, b % NBUF)

            @pl.when(i + 2 < CH)
            def _():
                wait_idx((b + 2) % IDEPTH)
                fire_gather((b + 2) % IDEPTH, (b + 2) % NBUF)

    wait_scatter((CH - 1) % IDEPTH, (CH - 1) % NBUF)
    plsc.subcore_barrier()

    @pl.when(s < 15)
    def _():
        row0 = pl.multiple_of(s * RPT, 8)
        pltpu.sync_copy(acc.at[pl.ds(row0, RPT)],
                        pre2.at[c, pl.ds(row0, RPT)])

    @pl.when(s == 15)
    def _():
        pltpu.sync_copy(acc.at[pl.ds(ROW0_LAST, RPT_LAST)],
                        pre2.at[c, pl.ds(ROW0_LAST, RPT_LAST)])


@functools.partial(
    pl.kernel,
    out_type=jax.ShapeDtypeStruct((NC, G, H), jnp.float32),
    mesh=_mesh,
    scratch_types=[
        pltpu.VMEM((PCH, PCHUNK), jnp.int32),
        pltpu.VMEM((PCHUNK, H), jnp.float32),
        pltpu.VMEM((GPT, H), jnp.float32),
        pltpu.VMEM_SHARED((GROWS, H), jnp.float32),
    ],
)
def _sc_pool(h2, batch_r, pooled, bidx, pbuf, zbuf, acc):
    c = lax.axis_index("c")
    s = lax.axis_index("s")

    @pl.loop(0, GPT)
    def _(r):
        for j in range(H // 16):
            zbuf[r, pl.ds(j * 16, 16)] = jnp.zeros((16,), jnp.float32)
    pltpu.sync_copy(zbuf, acc.at[pl.ds(s * GPT, GPT)])
    @pl.when(s == 0)
    def _():
        pltpu.sync_copy(zbuf.at[pl.ds(0, GROWS - G)],
                        acc.at[pl.ds(G, GROWS - G)])
    pltpu.sync_copy(batch_r.at[s], bidx)
    plsc.subcore_barrier()

    for k in range(PCH):
        ci = s * PCH + k

        @pl.when(ci < FULL_CHUNKS)
        def _():
            off = pl.multiple_of(ci * PCHUNK, 8)
            pltpu.sync_copy(h2.at[c, pl.ds(off, PCHUNK)], pbuf)

        @pl.when(ci == FULL_CHUNKS)
        def _():
            pltpu.sync_copy(h2.at[c, pl.ds(FULL_CHUNKS * PCHUNK, TAIL)],
                            pbuf.at[pl.ds(0, TAIL)])

        pltpu.sync_copy(pbuf, acc.at[bidx.at[k]], add=True)

    plsc.subcore_barrier()
    pltpu.sync_copy(acc.at[pl.ds(s * GPT, GPT)],
                    pooled.at[c, pl.ds(s * GPT, GPT)])


BN = 2000
NB = N // BN


def _tc_mlp_body(pre_ref, w1_ref, b1_ref, w2_ref, b2_ref, out_ref):
    x = jnp.concatenate([pre_ref[0], pre_ref[1]], axis=1)
    t = jnp.maximum(
        jnp.dot(x, w1_ref[...], preferred_element_type=jnp.float32)
        + b1_ref[...], 0.0)
    y = jnp.maximum(
        jnp.dot(t, w2_ref[...], preferred_element_type=jnp.float32)
        + b2_ref[...], 0.0)
    out_ref[0] = y[:, :H]
    out_ref[1] = y[:, H:]


def _tc_mlp(pre2, w1, b1, w2, b2):
    return pl.pallas_call(
        _tc_mlp_body,
        grid=(NB,),
        in_specs=[
            pl.BlockSpec((NC, BN, H), lambda i: (0, i, 0)),
            pl.BlockSpec((D, D), lambda i: (0, 0)),
            pl.BlockSpec((1, D), lambda i: (0, 0)),
            pl.BlockSpec((D, D), lambda i: (0, 0)),
            pl.BlockSpec((1, D), lambda i: (0, 0)),
        ],
        out_specs=pl.BlockSpec((NC, BN, H), lambda i: (0, i, 0)),
        out_shape=jax.ShapeDtypeStruct((NC, N, H), jnp.float32),
    )(pre2, w1, b1, w2, b2)


def _tc_mlp_pool_body(pre_ref, w1_ref, b1_ref, w2_ref, b2_ref, batch_ref,
                      out_ref, cnt_ref):
    i = pl.program_id(0)
    x = jnp.concatenate([pre_ref[0], pre_ref[1]], axis=1)
    t = jnp.maximum(
        jnp.dot(x, w1_ref[...], preferred_element_type=jnp.float32)
        + b1_ref[...], 0.0)
    y = jnp.maximum(
        jnp.dot(t, w2_ref[...], preferred_element_type=jnp.float32)
        + b2_ref[...], 0.0)
    gi = lax.broadcasted_iota(jnp.int32, (G, 1), 0)
    onehot = (batch_ref[0] == gi).astype(jnp.float32)
    psum = jnp.dot(onehot, y, preferred_element_type=jnp.float32)
    pcnt = jnp.sum(onehot, axis=1, keepdims=True)

    @pl.when(i == 0)
    def _():
        out_ref[...] = psum
        cnt_ref[...] = pcnt

    @pl.when(i > 0)
    def _():
        out_ref[...] += psum
        cnt_ref[...] += pcnt

    @pl.when(i == NB - 1)
    def _():
        out_ref[...] = out_ref[...] / jnp.maximum(cnt_ref[...], 1.0)


def _tc_mlp_pool(pre2, w1, b1, w2, b2, batch_3d):
    return pl.pallas_call(
        _tc_mlp_pool_body,
        grid=(NB,),
        in_specs=[
            pl.BlockSpec((NC, BN, H), lambda i: (0, i, 0)),
            pl.BlockSpec((D, D), lambda i: (0, 0)),
            pl.BlockSpec((1, D), lambda i: (0, 0)),
            pl.BlockSpec((D, D), lambda i: (0, 0)),
            pl.BlockSpec((1, D), lambda i: (0, 0)),
            pl.BlockSpec((1, 1, BN), lambda i: (i, 0, 0)),
        ],
        out_specs=pl.BlockSpec((G, D), lambda i: (0, 0)),
        out_shape=jax.ShapeDtypeStruct((G, D), jnp.float32),
        scratch_shapes=[pltpu.VMEM((G, 1), jnp.float32)],
    )(pre2, w1, b1, w2, b2, batch_3d)


def kernel(x, edge_index, batch, W1_0, b1_0, W2_0, b2_0, W1_1, b1_1, W2_1,
           b2_1, W1_2, b1_2, W2_2, b2_2):
    pad = E_PAD - E
    pad_vals = jnp.stack([jnp.zeros((pad,), jnp.int32),
                          jnp.full((pad,), N, jnp.int32)])
    edge_p = jnp.concatenate([edge_index, pad_vals], axis=1)
    edge_p = edge_p.reshape(2, NS, CH, CHUNK).transpose(1, 2, 0, 3)

    batch_3d = batch.reshape(NB, 1, BN)

    h2 = jnp.stack([x[:, :H], x[:, H:]])
    weights = [(W1_0, b1_0, W2_0, b2_0), (W1_1, b1_1, W2_1, b2_1)]

    for (w1, b1, w2, b2) in weights:
        pre2 = _sc_message(h2, edge_p)
        h2 = _tc_mlp(pre2, w1, b1.reshape(1, D), w2, b2.reshape(1, D))

    pre2 = _sc_message(h2, edge_p)
    return _tc_mlp_pool(pre2, W1_2, b1_2.reshape(1, D), W2_2,
                        b2_2.reshape(1, D), batch_3d)


# speedup vs baseline: 4.5836x; 1.0101x over previous
import functools

import jax
import jax.numpy as jnp
from jax import lax
from jax.experimental import pallas as pl
from jax.experimental.pallas import tpu as pltpu
from jax.experimental.pallas import tpu_sc as plsc

N = 10000
E = 160000
D = 256
H = 128
G = 128
NS = 16
NC = 2

CHUNK = 120
NBUF = 3
PAIRS = 7
GROUP = 42
CH = 84
EDGES_PER_TILE = CH * CHUNK
E_PAD = NS * EDGES_PER_TILE
NGROUPS = CH // GROUP
NROWS = N + 8

RPT = 624
RPT_LAST = 640
ROW0_LAST = 15 * RPT

PCH = 5
PCHUNK = 128
FULL_CHUNKS = N // PCHUNK
TAIL = N - FULL_CHUNKS * PCHUNK
GROWS = G + 8
GPT = G // NS

_mesh = plsc.VectorSubcoreMesh(core_axis_name="c", subcore_axis_name="s")


@functools.partial(
    pl.kernel,
    out_type=jax.ShapeDtypeStruct((NC, N, H), jnp.float32),
    mesh=_mesh,
    scratch_types=[
        pltpu.VMEM((PAIRS, 2, 2, CHUNK), jnp.int32),
        pltpu.VMEM((NBUF, CHUNK, H), jnp.float32),
        pltpu.VMEM_SHARED((NROWS, H), jnp.float32),
        pltpu.SemaphoreType.DMA((NBUF,)),
        pltpu.SemaphoreType.DMA((NBUF,)),
        pltpu.SemaphoreType.DMA((PAIRS,)),
        pltpu.SemaphoreType.DMA,
    ],
)
def _sc_message(h2, edge_p, pre2, idxv, rows, acc, gsem, ssem, isem,
                initsem):
    c = lax.axis_index("c")
    s = lax.axis_index("s")
    table = h2.at[c]

    @pl.when(s < 15)
    def _():
        row0 = pl.multiple_of(s * RPT, 8)
        pltpu.async_copy(table.at[pl.ds(row0, RPT)],
                         acc.at[pl.ds(row0, RPT)], initsem)

    @pl.when(s == 15)
    def _():
        pltpu.async_copy(table.at[pl.ds(ROW0_LAST, RPT_LAST)],
                         acc.at[pl.ds(ROW0_LAST, RPT_LAST)], initsem)

    def fire_pair(q, r):
        pltpu.async_copy(edge_p.at[s, pl.ds(2 * q, 2)], idxv.at[r],
                         isem.at[r])

    def wait_pair(r):
        pltpu.make_async_copy(edge_p.at[0, pl.ds(0, 2)], idxv.at[r],
                              isem.at[r]).wait()

    def gidx(b):
        return ((b // 2) % PAIRS, b % 2)

    def fire_gather(b, br):
        r, k = gidx(b)
        pltpu.async_copy(table.at[idxv.at[r, k, 0]], rows.at[br],
                         gsem.at[br])

    def wait_gather(b, br):
        r, k = gidx(b)
        pltpu.make_async_copy(table.at[idxv.at[r, k, 0]], rows.at[br],
                              gsem.at[br]).wait()

    def fire_scatter(b, br):
        r, k = gidx(b)
        pltpu.async_copy(rows.at[br], acc.at[idxv.at[r, k, 1]],
                         ssem.at[br], add=True)

    def wait_scatter(b, br):
        r, k = gidx(b)
        pltpu.make_async_copy(rows.at[br], acc.at[idxv.at[r, k, 1]],
                              ssem.at[br]).wait()

    for q in range(PAIRS - 1):
        fire_pair(q, q)
    wait_pair(0)
    fire_gather(0, 0)
    fire_gather(1, 1)

    @pl.when(s < 15)
    def _():
        pltpu.make_async_copy(table.at[pl.ds(0, RPT)],
                              acc.at[pl.ds(0, RPT)], initsem).wait()

    @pl.when(s == 15)
    def _():
        pltpu.make_async_copy(table.at[pl.ds(0, RPT_LAST)],
                              acc.at[pl.ds(0, RPT_LAST)], initsem).wait()
    plsc.subcore_barrier()

    @pl.loop(0, NGROUPS)
    def _(g):
        for b in range(GROUP):
            i = g * GROUP + b

            @pl.when(i >= 1)
            def _():
                wait_scatter((b - 1) % GROUP, (b - 1) % NBUF)

            if b % 2 == 1:
                @pl.when(i + 11 < CH)
                def _():
                    fire_pair((i + 11) // 2, ((b + 11) // 2) % PAIRS)

            wait_gather(b, b % NBUF)
            fire_scatter(b, b % NBUF)

            @pl.when(i + 2 < CH)
            def _():
                if b % 2 == 0:
                    wait_pair(((b + 2) // 2) % PAIRS)
                fire_gather((b + 2) % GROUP, (b + 2) % NBUF)

    wait_scatter((CH - 1) % GROUP, (CH - 1) % NBUF)
    plsc.subcore_barrier()

    @pl.when(s < 15)
    def _():
        row0 = pl.multiple_of(s * RPT, 8)
        pltpu.sync_copy(acc.at[pl.ds(row0, RPT)],
                        pre2.at[c, pl.ds(row0, RPT)])

    @pl.when(s == 15)
    def _():
        pltpu.sync_copy(acc.at[pl.ds(ROW0_LAST, RPT_LAST)],
                        pre2.at[c, pl.ds(ROW0_LAST, RPT_LAST)])


@functools.partial(
    pl.kernel,
    out_type=jax.ShapeDtypeStruct((NC, G, H), jnp.float32),
    mesh=_mesh,
    scratch_types=[
        pltpu.VMEM((PCH, PCHUNK), jnp.int32),
        pltpu.VMEM((PCHUNK, H), jnp.float32),
        pltpu.VMEM((GPT, H), jnp.float32),
        pltpu.VMEM_SHARED((GROWS, H), jnp.float32),
    ],
)
def _sc_pool(h2, batch_r, pooled, bidx, pbuf, zbuf, acc):
    c = lax.axis_index("c")
    s = lax.axis_index("s")

    @pl.loop(0, GPT)
    def _(r):
        for j in range(H // 16):
            zbuf[r, pl.ds(j * 16, 16)] = jnp.zeros((16,), jnp.float32)
    pltpu.sync_copy(zbuf, acc.at[pl.ds(s * GPT, GPT)])
    @pl.when(s == 0)
    def _():
        pltpu.sync_copy(zbuf.at[pl.ds(0, GROWS - G)],
                        acc.at[pl.ds(G, GROWS - G)])
    pltpu.sync_copy(batch_r.at[s], bidx)
    plsc.subcore_barrier()

    for k in range(PCH):
        ci = s * PCH + k

        @pl.when(ci < FULL_CHUNKS)
        def _():
            off = pl.multiple_of(ci * PCHUNK, 8)
            pltpu.sync_copy(h2.at[c, pl.ds(off, PCHUNK)], pbuf)

        @pl.when(ci == FULL_CHUNKS)
        def _():
            pltpu.sync_copy(h2.at[c, pl.ds(FULL_CHUNKS * PCHUNK, TAIL)],
                            pbuf.at[pl.ds(0, TAIL)])

        pltpu.sync_copy(pbuf, acc.at[bidx.at[k]], add=True)

    plsc.subcore_barrier()
    pltpu.sync_copy(acc.at[pl.ds(s * GPT, GPT)],
                    pooled.at[c, pl.ds(s * GPT, GPT)])


BN = 2000
NB = N // BN


def _tc_mlp_body(pre_ref, w1_ref, b1_ref, w2_ref, b2_ref, out_ref):
    x = jnp.concatenate([pre_ref[0], pre_ref[1]], axis=1)
    t = jnp.maximum(
        jnp.dot(x, w1_ref[...], preferred_element_type=jnp.float32)
        + b1_ref[...], 0.0)
    y = jnp.maximum(
        jnp.dot(t, w2_ref[...], preferred_element_type=jnp.float32)
        + b2_ref[...], 0.0)
    out_ref[0] = y[:, :H]
    out_ref[1] = y[:, H:]


def _tc_mlp(pre2, w1, b1, w2, b2):
    return pl.pallas_call(
        _tc_mlp_body,
        grid=(NB,),
        in_specs=[
            pl.BlockSpec((NC, BN, H), lambda i: (0, i, 0)),
            pl.BlockSpec((D, D), lambda i: (0, 0)),
            pl.BlockSpec((1, D), lambda i: (0, 0)),
            pl.BlockSpec((D, D), lambda i: (0, 0)),
            pl.BlockSpec((1, D), lambda i: (0, 0)),
        ],
        out_specs=pl.BlockSpec((NC, BN, H), lambda i: (0, i, 0)),
        out_shape=jax.ShapeDtypeStruct((NC, N, H), jnp.float32),
    )(pre2, w1, b1, w2, b2)


def _tc_mlp_pool_body(pre_ref, w1_ref, b1_ref, w2_ref, b2_ref, batch_ref,
                      out_ref, cnt_ref):
    i = pl.program_id(0)
    x = jnp.concatenate([pre_ref[0], pre_ref[1]], axis=1)
    t = jnp.maximum(
        jnp.dot(x, w1_ref[...], preferred_element_type=jnp.float32)
        + b1_ref[...], 0.0)
    y = jnp.maximum(
        jnp.dot(t, w2_ref[...], preferred_element_type=jnp.float32)
        + b2_ref[...], 0.0)
    gi = lax.broadcasted_iota(jnp.int32, (G, 1), 0)
    onehot = (batch_ref[0] == gi).astype(jnp.float32)
    psum = jnp.dot(onehot, y, preferred_element_type=jnp.float32)
    pcnt = jnp.sum(onehot, axis=1, keepdims=True)

    @pl.when(i == 0)
    def _():
        out_ref[...] = psum
        cnt_ref[...] = pcnt

    @pl.when(i > 0)
    def _():
        out_ref[...] += psum
        cnt_ref[...] += pcnt

    @pl.when(i == NB - 1)
    def _():
        out_ref[...] = out_ref[...] / jnp.maximum(cnt_ref[...], 1.0)


def _tc_mlp_pool(pre2, w1, b1, w2, b2, batch_3d):
    return pl.pallas_call(
        _tc_mlp_pool_body,
        grid=(NB,),
        in_specs=[
            pl.BlockSpec((NC, BN, H), lambda i: (0, i, 0)),
            pl.BlockSpec((D, D), lambda i: (0, 0)),
            pl.BlockSpec((1, D), lambda i: (0, 0)),
            pl.BlockSpec((D, D), lambda i: (0, 0)),
            pl.BlockSpec((1, D), lambda i: (0, 0)),
            pl.BlockSpec((1, 1, BN), lambda i: (i, 0, 0)),
        ],
        out_specs=pl.BlockSpec((G, D), lambda i: (0, 0)),
        out_shape=jax.ShapeDtypeStruct((G, D), jnp.float32),
        scratch_shapes=[pltpu.VMEM((G, 1), jnp.float32)],
    )(pre2, w1, b1, w2, b2, batch_3d)


def kernel(x, edge_index, batch, W1_0, b1_0, W2_0, b2_0, W1_1, b1_1, W2_1,
           b2_1, W1_2, b1_2, W2_2, b2_2):
    pad = E_PAD - E
    pad_vals = jnp.stack([jnp.zeros((pad,), jnp.int32),
                          jnp.full((pad,), N, jnp.int32)])
    edge_p = jnp.concatenate([edge_index, pad_vals], axis=1)
    edge_p = edge_p.reshape(2, NS, CH, CHUNK).transpose(1, 2, 0, 3)

    batch_3d = batch.reshape(NB, 1, BN)

    h2 = jnp.stack([x[:, :H], x[:, H:]])
    weights = [(W1_0, b1_0, W2_0, b2_0), (W1_1, b1_1, W2_1, b2_1)]

    for (w1, b1, w2, b2) in weights:
        pre2 = _sc_message(h2, edge_p)
        h2 = _tc_mlp(pre2, w1, b1.reshape(1, D), w2, b2.reshape(1, D))

    pre2 = _sc_message(h2, edge_p)
    return _tc_mlp_pool(pre2, W1_2, b1_2.reshape(1, D), W2_2,
                        b2_2.reshape(1, D), batch_3d)
